# Initial kernel scaffold; baseline (speedup 1.0000x reference)
import jax
import jax.numpy as jnp
from jax.experimental import pallas as pl


def kernel(temporal_block, temporal_masked_idx, temporal_revert_idx, mask_token_param, temporal_mod_emb_table):
    raise NotImplementedError("write your pallas kernel here")


# speedup vs baseline: 4.4529x; 4.4529x over previous
import numpy as np
import jax
import jax.numpy as jnp
from jax.experimental import pallas as pl
from jax.experimental.pallas import tpu as pltpu

_B = 16
_S = 512
_M_VALID = 8
_N_MASKED = 8
_D = 256
_N_MOD = 1 + _M_VALID + _N_MASKED

_SC = 128


def _pos_encoding_np(seq_len, d_model):
    pos = np.arange(seq_len, dtype=np.float32)[:, None]
    div = np.exp(np.arange(0, d_model, 2, dtype=np.float32) * (-np.log(10000.0) / d_model))
    pe = np.zeros((seq_len, d_model), dtype=np.float32)
    pe[:, 0::2] = np.sin(pos * div)
    pe[:, 1::2] = np.cos(pos * div)
    return pe


_PE = _pos_encoding_np(_S, _D)


def _body(tb_ref, idx_ref, mt_ref, emb_ref, pe_ref, out_ref):
    pe = pe_ref[...]
    idx = idx_ref[0]
    mt = mt_ref[0]
    out_ref[0, :, 0, :] = tb_ref[0, :, 0, :] + pe + emb_ref[0]
    for m in range(_M_VALID + _N_MASKED):
        im = idx[:, m][:, None]
        acc = jnp.broadcast_to(mt[None, :], (_SC, _D))
        for k in range(_M_VALID):
            acc = jnp.where(im == k, tb_ref[0, :, 1 + k, :], acc)
        out_ref[0, :, 1 + m, :] = acc + pe + emb_ref[1 + m]


def kernel(temporal_block, temporal_masked_idx, temporal_revert_idx,
           mask_token_param, temporal_mod_emb_table):
    del temporal_masked_idx
    b, s, _, d = temporal_block.shape
    pe = jnp.asarray(_PE)
    mt = mask_token_param.reshape(1, _D)
    grid = (b, s // _SC)
    out = pl.pallas_call(
        _body,
        grid=grid,
        in_specs=[
            pl.BlockSpec((1, _SC, 1 + _M_VALID, _D), lambda i, j: (i, j, 0, 0)),
            pl.BlockSpec((1, _SC, _M_VALID + _N_MASKED), lambda i, j: (i, j, 0)),
            pl.BlockSpec((1, _D), lambda i, j: (0, 0)),
            pl.BlockSpec((_N_MOD, _D), lambda i, j: (0, 0)),
            pl.BlockSpec((_SC, _D), lambda i, j: (j, 0)),
        ],
        out_specs=pl.BlockSpec((1, _SC, _N_MOD, _D), lambda i, j: (i, j, 0, 0)),
        out_shape=jax.ShapeDtypeStruct((b, s, _N_MOD, d), jnp.float32),
        compiler_params=pltpu.CompilerParams(
            dimension_semantics=("parallel", "parallel"),
        ),
    )(temporal_block, temporal_revert_idx, mt, temporal_mod_emb_table, pe)
    return out
